# baseline (speedup 1.0000x reference)
import functools

import jax
import jax.numpy as jnp
from jax.experimental import pallas as pl

G, IN_DIM, OUT_DIM, COLS = 4, 4096, 4096, 256
BK = 512


def _body(x_ref, w_ref, b_ref, o_ref):
    k = pl.program_id(1)
    acc = jax.lax.dot_general(
        w_ref[0], x_ref[0],
        dimension_numbers=(((0,), (0,)), ((), ())),
        preferred_element_type=jnp.float32,
    )

    @pl.when(k == 0)
    def _():
        o_ref[0] = acc + b_ref[0]

    @pl.when(k > 0)
    def _():
        o_ref[0] += acc


@functools.partial(jax.jit, static_argnames=("interpret",))
def kernel(inputs, W, b, interpret=False):
    nk = IN_DIM // BK
    return pl.pallas_call(
        _body,
        grid=(G, nk),
        in_specs=[
            pl.BlockSpec((1, BK, COLS), lambda g, k: (g, k, 0)),
            pl.BlockSpec((1, BK, OUT_DIM), lambda g, k: (g, k, 0)),
            pl.BlockSpec((1, OUT_DIM, 1), lambda g, k: (g, 0, 0)),
        ],
        out_specs=pl.BlockSpec((1, OUT_DIM, COLS), lambda g, k: (g, 0, 0)),
        out_shape=jax.ShapeDtypeStruct((G, OUT_DIM, COLS), jnp.float32),
        interpret=interpret,
    )(inputs, W, b)


# speedup vs baseline: 1.0895x; 1.0895x over previous
import functools

import jax
import jax.numpy as jnp
from jax.experimental import pallas as pl

G, IN_DIM, OUT_DIM, COLS = 4, 4096, 4096, 256
BK = 1024


def _body(x_ref, w_ref, b_ref, o_ref):
    k = pl.program_id(1)
    acc = jax.lax.dot_general(
        w_ref[0], x_ref[0],
        dimension_numbers=(((0,), (0,)), ((), ())),
        preferred_element_type=jnp.float32,
    )

    @pl.when(k == 0)
    def _():
        o_ref[0] = acc + b_ref[0]

    @pl.when(k > 0)
    def _():
        o_ref[0] += acc


@functools.partial(jax.jit, static_argnames=("interpret",))
def kernel(inputs, W, b, interpret=False):
    nk = IN_DIM // BK
    return pl.pallas_call(
        _body,
        grid=(G, nk),
        in_specs=[
            pl.BlockSpec((1, BK, COLS), lambda g, k: (g, k, 0)),
            pl.BlockSpec((1, BK, OUT_DIM), lambda g, k: (g, k, 0)),
            pl.BlockSpec((1, OUT_DIM, 1), lambda g, k: (g, 0, 0)),
        ],
        out_specs=pl.BlockSpec((1, OUT_DIM, COLS), lambda g, k: (g, 0, 0)),
        out_shape=jax.ShapeDtypeStruct((G, OUT_DIM, COLS), jnp.float32),
        interpret=interpret,
    )(inputs, W, b)
